# baseline (speedup 1.0000x reference)
import jax
import jax.numpy as jnp
from jax.experimental import pallas as pl
from jax.experimental.pallas import tpu as pltpu

N = 10000
BM = 400
NI = N // BM
W = 16


def _layer1_body(adj_ref, x_ref, w1_ref, b1_ref, w2_ref,
                 s2_ref, adjq_ref, rs_ref, bmax_ref):
    a = adj_ref[...]
    rmax = jnp.maximum(jnp.max(jnp.abs(a), axis=1, keepdims=True), 1e-30)
    adjq_ref[0] = jnp.round(a * (127.0 / rmax)).astype(jnp.int8)
    rs_ref[...] = rmax * (1.0 / 127.0)
    acc = jnp.dot(a.astype(jnp.bfloat16), x_ref[...],
                  preferred_element_type=jnp.float32)
    h = jnp.dot(acc, w1_ref[...], preferred_element_type=jnp.float32)
    h = jnp.maximum(h + b1_ref[...], 0.0)
    s2 = jnp.dot(h, w2_ref[...], preferred_element_type=jnp.float32)
    s2_ref[...] = s2
    bmax_ref[0] = jnp.max(jnp.abs(s2), axis=0, keepdims=True)


def _mids_body(adjq_ref, rs_ref, s2_ref, bmax_ref, ball_ref, wall_ref,
               out_ref, sf_ref, sq_ref, cs_ref, cmax_ref):
    l = pl.program_id(0)
    i = pl.program_id(1)

    @pl.when(jnp.logical_and(l == 0, i == 0))
    def _():
        cm = jnp.max(bmax_ref[...], axis=0)
        cs = jnp.maximum(cm, 1e-30) * (1.0 / 127.0)
        cs_ref[...] = cs
        sq_ref[...] = jnp.round(s2_ref[...] * (1.0 / cs)).astype(jnp.int8)
        cmax_ref[...] = jnp.zeros_like(cmax_ref)

    @pl.when(jnp.logical_and(l > 0, i == 0))
    def _():
        cs = jnp.maximum(cmax_ref[...], 1e-30) * (1.0 / 127.0)
        cs_ref[...] = cs
        sq_ref[...] = jnp.round(sf_ref[...] * (1.0 / cs)).astype(jnp.int8)
        cmax_ref[...] = jnp.zeros_like(cmax_ref)

    acc = jnp.dot(adjq_ref[0], sq_ref[...], preferred_element_type=jnp.int32)
    z = acc.astype(jnp.float32) * rs_ref[...] * cs_ref[...] + ball_ref[l]
    h = jnp.maximum(z, 0.0)
    s_next = jnp.dot(h, wall_ref[l], preferred_element_type=jnp.float32)

    @pl.when(l < 2)
    def _():
        sf_ref[pl.ds(i * BM, BM), :] = s_next
        cmax_ref[...] = jnp.maximum(
            cmax_ref[...], jnp.max(jnp.abs(s_next), axis=0, keepdims=True))
        out_ref[0] = s_next

    @pl.when(l == 2)
    def _():
        m = jnp.max(z, axis=1, keepdims=True)
        zz = z - m
        lse = jnp.log(jnp.sum(jnp.exp(zz), axis=1, keepdims=True))
        out_ref[0] = zz - lse


def _full_spec(shape):
    return pl.BlockSpec(shape, lambda *_: tuple(0 for _ in shape))


_CP1 = pltpu.CompilerParams(dimension_semantics=("arbitrary",))
_CP2 = pltpu.CompilerParams(dimension_semantics=("arbitrary", "arbitrary"))


def _pad_to(a, shape):
    return jnp.zeros(shape, a.dtype).at[tuple(slice(0, d) for d in a.shape)].set(a)


@jax.jit
def kernel(x, adj, W1, b1, W2, b2, W3, b3, W4, b4):
    s2, adjq, rs, bmax = pl.pallas_call(
        _layer1_body,
        grid=(NI,),
        in_specs=[pl.BlockSpec((BM, N), lambda i: (i, 0)),
                  _full_spec(x.shape), _full_spec(W1.shape),
                  _full_spec((1, W1.shape[1])), _full_spec(W2.shape)],
        out_specs=[pl.BlockSpec((BM, W), lambda i: (i, 0)),
                   pl.BlockSpec((1, BM, N), lambda i: (i, 0, 0)),
                   pl.BlockSpec((BM, 1), lambda i: (i, 0)),
                   pl.BlockSpec((1, 1, W), lambda i: (i, 0, 0))],
        out_shape=[jax.ShapeDtypeStruct((N, W), jnp.float32),
                   jax.ShapeDtypeStruct((NI, BM, N), jnp.int8),
                   jax.ShapeDtypeStruct((N, 1), jnp.float32),
                   jax.ShapeDtypeStruct((NI, 1, W), jnp.float32)],
        compiler_params=_CP1,
    )(adj, x.astype(jnp.bfloat16), W1, b1.reshape(1, -1), W2)

    b_all = jnp.stack([b2.reshape(1, W),
                       _pad_to(b3.reshape(1, -1), (1, W)),
                       b4.reshape(1, W)])
    w_all = jnp.stack([_pad_to(W3, (W, W)), _pad_to(W4, (W, W)),
                       jnp.zeros((W, W), jnp.float32)])

    return pl.pallas_call(
        _mids_body,
        grid=(3, NI),
        in_specs=[pl.BlockSpec((1, BM, N), lambda l, i: (i, 0, 0)),
                  pl.BlockSpec((BM, 1), lambda l, i: (i, 0)),
                  _full_spec((N, W)), _full_spec((NI, 1, W)),
                  _full_spec((3, 1, W)), _full_spec((3, W, W))],
        out_specs=pl.BlockSpec((1, BM, W), lambda l, i: (l, i, 0)),
        out_shape=jax.ShapeDtypeStruct((3, N, W), jnp.float32),
        scratch_shapes=[pltpu.VMEM((N, W), jnp.float32),
                        pltpu.VMEM((N, W), jnp.int8),
                        pltpu.VMEM((1, W), jnp.float32),
                        pltpu.VMEM((1, W), jnp.float32)],
        compiler_params=_CP2,
    )(adjq, rs, s2, bmax, b_all, w_all)[2]
